# Initial kernel scaffold; baseline (speedup 1.0000x reference)
import jax
import jax.numpy as jnp
from jax.experimental import pallas as pl


def kernel(x, edge_index, edge_attr, W1, b1, g1, be1, W2, b2, g2, be2, W3, b3):
    raise NotImplementedError("write your pallas kernel here")


# speedup vs baseline: 2.2126x; 2.2126x over previous
import functools

import jax
import jax.numpy as jnp
from jax import lax
from jax.experimental import pallas as pl
from jax.experimental.pallas import tpu as pltpu
from jax.experimental.pallas import tpu_sc as plsc


def _pre_body(x_ref, w_ref, xs_ref, xr_ref):
    xsr = jnp.dot(x_ref[...], w_ref[...], preferred_element_type=jnp.float32)
    h = xs_ref.shape[1]
    xs_ref[...] = xsr[:, :h]
    xr_ref[...] = xsr[:, h:]


def _precompute_tables(x, w_cat, hid):
    n = x.shape[0]
    return pl.pallas_call(
        _pre_body,
        out_shape=[
            jax.ShapeDtypeStruct((n, hid), jnp.float32),
            jax.ShapeDtypeStruct((n, hid), jnp.float32),
        ],
    )(x, w_cat)


def _make_sc_gather(e_total, hid, nc, ns, chunk):
    nw = nc * ns
    ew = e_total // nw
    nchunk = ew // chunk
    mesh = plsc.VectorSubcoreMesh(core_axis_name="c", subcore_axis_name="s",
                                  num_cores=nc, num_subcores=ns)

    @functools.partial(
        pl.kernel,
        out_type=jax.ShapeDtypeStruct((e_total, hid), jnp.float32),
        mesh=mesh,
        scratch_types=[
            pltpu.VMEM((chunk,), jnp.int32),
            pltpu.VMEM((chunk,), jnp.int32),
            pltpu.VMEM((chunk, hid), jnp.float32),
            pltpu.VMEM((chunk, hid), jnp.float32),
            pltpu.SemaphoreType.DMA,
        ],
        compiler_params=pltpu.CompilerParams(use_tc_tiling_on_sc=False),
    )
    def sc_gather(xs_hbm, xr_hbm, si_hbm, ri_hbm, g_hbm,
                  si_v, ri_v, a_v, b_v, sem):
        wid = lax.axis_index("s") * nc + lax.axis_index("c")
        base = wid * ew

        def do_chunk(i, carry):
            off = base + i * chunk
            pltpu.sync_copy(si_hbm.at[pl.ds(off, chunk)], si_v)
            pltpu.sync_copy(ri_hbm.at[pl.ds(off, chunk)], ri_v)
            cp_a = pltpu.async_copy(xs_hbm.at[si_v], a_v, sem)
            cp_b = pltpu.async_copy(xr_hbm.at[ri_v], b_v, sem)
            cp_a.wait()
            cp_b.wait()

            def add_row(r, c2):
                for j in range(hid // 16):
                    sl = pl.ds(j * 16, 16)
                    a_v[r, sl] = a_v[r, sl] + b_v[r, sl]
                return c2

            lax.fori_loop(0, chunk, add_row, 0, unroll=2)
            pltpu.sync_copy(a_v, g_hbm.at[pl.ds(off, chunk)])
            return carry

        lax.fori_loop(0, nchunk, do_chunk, 0)

    return sc_gather


def _ln(h, g, b, eps=1e-5):
    m = jnp.mean(h, axis=-1, keepdims=True)
    v = jnp.mean((h - m) * (h - m), axis=-1, keepdims=True)
    return (h - m) * lax.rsqrt(v + eps) * g + b


def _mlp_body(g_ref, ea_ref, w1e_ref, b1_ref, g1_ref, be1_ref,
              w2_ref, b2_ref, g2_ref, be2_ref, w3_ref, b3_ref, o_ref):
    h = (g_ref[...]
         + jnp.dot(ea_ref[...], w1e_ref[...], preferred_element_type=jnp.float32)
         + b1_ref[...])
    h = _ln(h, g1_ref[...], be1_ref[...])
    h = h * jax.nn.sigmoid(h)
    h = jnp.dot(h, w2_ref[...], preferred_element_type=jnp.float32) + b2_ref[...]
    h = _ln(h, g2_ref[...], be2_ref[...])
    h = h * jax.nn.sigmoid(h)
    o_ref[...] = (jnp.dot(h, w3_ref[...], preferred_element_type=jnp.float32)
                  + b3_ref[...])


def _mlp_call(g, ea, w1e, b1, g1, be1, w2, b2, g2, be2, w3, b3, block):
    e_total, hid = g.shape
    d_edge = ea.shape[1]
    out_dim = w3.shape[1]
    grid = (e_total // block,)

    def _blk(shape):
        return pl.BlockSpec(shape, lambda i: (i, 0))

    def _full(shape):
        return pl.BlockSpec(shape, lambda i: (0, 0))

    return pl.pallas_call(
        _mlp_body,
        grid=grid,
        in_specs=[
            _blk((block, hid)),
            _blk((block, d_edge)),
            _full(w1e.shape), _full(b1.shape), _full(g1.shape), _full(be1.shape),
            _full(w2.shape), _full(b2.shape), _full(g2.shape), _full(be2.shape),
            _full(w3.shape), _full(b3.shape),
        ],
        out_specs=_blk((block, out_dim)),
        out_shape=jax.ShapeDtypeStruct((e_total, out_dim), jnp.float32),
    )(g, ea, w1e, b1, g1, be1, w2, b2, g2, be2, w3, b3)


def kernel(x, edge_index, edge_attr, W1, b1, g1, be1, W2, b2, g2, be2, W3, b3):
    n, d_feat = x.shape
    e_total, d_edge = edge_attr.shape
    hid = W2.shape[0]

    try:
        info = plsc.get_sparse_core_info()
        nc, ns = info.num_cores, info.num_subcores
    except Exception:
        nc, ns = 2, 16

    w_cat = jnp.concatenate([W1[:d_feat], W1[d_feat:2 * d_feat]], axis=1)
    w1e = W1[2 * d_feat:]

    xs, xr = _precompute_tables(x, w_cat, hid)

    g = _make_sc_gather(e_total, hid, nc, ns, chunk=80)(
        xs, xr, edge_index[0], edge_index[1])

    row = lambda v: v.reshape(1, -1)
    return _mlp_call(g, edge_attr, w1e, row(b1), row(g1), row(be1),
                     W2, row(b2), row(g2), row(be2), W3, row(b3), block=2000)


# speedup vs baseline: 3.1852x; 1.4396x over previous
import functools

import jax
import jax.numpy as jnp
from jax import lax
from jax.experimental import pallas as pl
from jax.experimental.pallas import tpu as pltpu
from jax.experimental.pallas import tpu_sc as plsc


def _pre_body(x_ref, w_ref, xs_ref, xr_ref):
    xsr = jnp.dot(x_ref[...], w_ref[...], preferred_element_type=jnp.float32)
    h = xs_ref.shape[1]
    xs_ref[...] = xsr[:, :h]
    xr_ref[...] = xsr[:, h:]


def _precompute_tables(x, w_cat, hid):
    n = x.shape[0]
    return pl.pallas_call(
        _pre_body,
        out_shape=[
            jax.ShapeDtypeStruct((n, hid), jnp.float32),
            jax.ShapeDtypeStruct((n, hid), jnp.float32),
        ],
    )(x, w_cat)


def _make_sc_gather(e_total, hid, nc, ns, chunk, nbuf):
    nw = nc * ns
    ew = e_total // nw
    nchunk = ew // chunk
    assert nchunk % nbuf == 0
    nround = nchunk // nbuf
    mesh = plsc.VectorSubcoreMesh(core_axis_name="c", subcore_axis_name="s",
                                  num_cores=nc, num_subcores=ns)

    @functools.partial(
        pl.kernel,
        out_type=jax.ShapeDtypeStruct((e_total, hid), jnp.float32),
        mesh=mesh,
        scratch_types=[
            pltpu.VMEM((ew,), jnp.int32),
            pltpu.VMEM((ew,), jnp.int32),
            pltpu.VMEM((nbuf, chunk, hid), jnp.float32),
            pltpu.VMEM((nbuf, chunk, hid), jnp.float32),
            pltpu.VMEM((nbuf, chunk, hid), jnp.float32),
            pltpu.SemaphoreType.DMA((nbuf,)),
            pltpu.SemaphoreType.DMA((nbuf,)),
        ],
        compiler_params=pltpu.CompilerParams(use_tc_tiling_on_sc=False),
    )
    def sc_gather(xs_hbm, xr_hbm, si_hbm, ri_hbm, g_hbm,
                  si_v, ri_v, a_v, b_v, o_v, gsem, wsem):
        wid = lax.axis_index("s") * nc + lax.axis_index("c")
        base = wid * ew
        pltpu.sync_copy(si_hbm.at[pl.ds(base, ew)], si_v)
        pltpu.sync_copy(ri_hbm.at[pl.ds(base, ew)], ri_v)

        def start_gather(i, b):
            sl = pl.ds(i * chunk, chunk)
            pltpu.async_copy(xs_hbm.at[si_v.at[sl]], a_v.at[b], gsem.at[b])
            pltpu.async_copy(xr_hbm.at[ri_v.at[sl]], b_v.at[b], gsem.at[b])

        for b in range(nbuf):
            start_gather(b, b)

        def do_round(k, carry):
            for b in range(nbuf):
                i = k * nbuf + b
                pltpu.make_async_copy(
                    xs_hbm.at[pl.ds(0, chunk)], a_v.at[b], gsem.at[b]).wait()
                pltpu.make_async_copy(
                    xr_hbm.at[pl.ds(0, chunk)], b_v.at[b], gsem.at[b]).wait()

                @pl.when(k > 0)
                def _():
                    pltpu.make_async_copy(
                        o_v.at[b], g_hbm.at[pl.ds(base, chunk)],
                        wsem.at[b]).wait()

                def add_row(r, c2):
                    for j in range(hid // 16):
                        s16 = pl.ds(j * 16, 16)
                        o_v[b, r, s16] = a_v[b, r, s16] + b_v[b, r, s16]
                    return c2

                lax.fori_loop(0, chunk, add_row, 0, unroll=4)

                pltpu.async_copy(
                    o_v.at[b], g_hbm.at[pl.ds(base + i * chunk, chunk)],
                    wsem.at[b])

                @pl.when(i + nbuf < nchunk)
                def _():
                    start_gather(i + nbuf, b)
            return carry

        lax.fori_loop(0, nround, do_round, 0)

        for b in range(nbuf):
            pltpu.make_async_copy(
                o_v.at[b], g_hbm.at[pl.ds(base, chunk)], wsem.at[b]).wait()

    return sc_gather


def _ln(h, g, b, eps=1e-5):
    m = jnp.mean(h, axis=-1, keepdims=True)
    v = jnp.mean((h - m) * (h - m), axis=-1, keepdims=True)
    return (h - m) * lax.rsqrt(v + eps) * g + b


def _mlp_body(g_ref, ea_ref, w1e_ref, b1_ref, g1_ref, be1_ref,
              w2_ref, b2_ref, g2_ref, be2_ref, w3_ref, b3_ref, o_ref):
    h = (g_ref[...]
         + jnp.dot(ea_ref[...], w1e_ref[...], preferred_element_type=jnp.float32)
         + b1_ref[...])
    h = _ln(h, g1_ref[...], be1_ref[...])
    h = h * jax.nn.sigmoid(h)
    h = jnp.dot(h, w2_ref[...], preferred_element_type=jnp.float32) + b2_ref[...]
    h = _ln(h, g2_ref[...], be2_ref[...])
    h = h * jax.nn.sigmoid(h)
    o_ref[...] = (jnp.dot(h, w3_ref[...], preferred_element_type=jnp.float32)
                  + b3_ref[...])


def _mlp_call(g, ea, w1e, b1, g1, be1, w2, b2, g2, be2, w3, b3, block):
    e_total, hid = g.shape
    d_edge = ea.shape[1]
    out_dim = w3.shape[1]
    grid = (e_total // block,)

    def _blk(shape):
        return pl.BlockSpec(shape, lambda i: (i, 0))

    def _full(shape):
        return pl.BlockSpec(shape, lambda i: (0, 0))

    return pl.pallas_call(
        _mlp_body,
        grid=grid,
        in_specs=[
            _blk((block, hid)),
            _blk((block, d_edge)),
            _full(w1e.shape), _full(b1.shape), _full(g1.shape), _full(be1.shape),
            _full(w2.shape), _full(b2.shape), _full(g2.shape), _full(be2.shape),
            _full(w3.shape), _full(b3.shape),
        ],
        out_specs=_blk((block, out_dim)),
        out_shape=jax.ShapeDtypeStruct((e_total, out_dim), jnp.float32),
    )(g, ea, w1e, b1, g1, be1, w2, b2, g2, be2, w3, b3)


def kernel(x, edge_index, edge_attr, W1, b1, g1, be1, W2, b2, g2, be2, W3, b3):
    n, d_feat = x.shape
    e_total, d_edge = edge_attr.shape
    hid = W2.shape[0]

    try:
        info = plsc.get_sparse_core_info()
        nc, ns = info.num_cores, info.num_subcores
    except Exception:
        nc, ns = 2, 16

    w_cat = jnp.concatenate([W1[:d_feat], W1[d_feat:2 * d_feat]], axis=1)
    w1e = W1[2 * d_feat:]

    xs, xr = _precompute_tables(x, w_cat, hid)

    g = _make_sc_gather(e_total, hid, nc, ns, chunk=80, nbuf=5)(
        xs, xr, edge_index[0], edge_index[1])

    row = lambda v: v.reshape(1, -1)
    return _mlp_call(g, edge_attr, w1e, row(b1), row(g1), row(be1),
                     W2, row(b2), row(g2), row(be2), W3, row(b3), block=2000)
